# Initial kernel scaffold; baseline (speedup 1.0000x reference)
import jax
import jax.numpy as jnp
from jax.experimental import pallas as pl


def kernel(y, x, B, A, u_grid):
    raise NotImplementedError("write your pallas kernel here")


# speedup vs baseline: 1.3346x; 1.3346x over previous
import functools

import jax
import jax.numpy as jnp
from jax import lax
from jax.experimental import pallas as pl
from jax.experimental.pallas import tpu as pltpu
from jax.experimental.pallas import tpu_sc as plsc

N = 4096
D_X = 128
D_Y = 2
K = 2500
K_PAD = 2560
N_BLK = 256
N_GRID = N // N_BLK
A_PAD = 2.0e18
U_COLS = 128


def _dist_argmin_body(x_ref, w_ref, a_ref, y_ref, idx_ref):
    s = jnp.dot(x_ref[...], w_ref[...], preferred_element_type=jnp.float32)
    s = s + a_ref[...]
    d0 = s[:, :K_PAD] - y_ref[:, 0:1]
    d1 = s[:, K_PAD:] - y_ref[:, 1:2]
    d2 = d0 * d0 + d1 * d1
    m = jnp.min(d2, axis=1, keepdims=True)
    k_iota = lax.broadcasted_iota(jnp.int32, d2.shape, 1)
    idx = jnp.min(jnp.where(d2 == m, k_iota, K_PAD), axis=1)
    idx_ref[0, 0, :] = idx


def _argmin_indices(x, w, a, y):
    return pl.pallas_call(
        _dist_argmin_body,
        grid=(N_GRID,),
        in_specs=[
            pl.BlockSpec((N_BLK, D_X), lambda i: (i, 0)),
            pl.BlockSpec((D_X, 2 * K_PAD), lambda i: (0, 0)),
            pl.BlockSpec((1, 2 * K_PAD), lambda i: (0, 0)),
            pl.BlockSpec((N_BLK, D_Y), lambda i: (i, 0)),
        ],
        out_specs=pl.BlockSpec((1, 1, N_BLK), lambda i: (i, 0, 0)),
        out_shape=jax.ShapeDtypeStruct((N_GRID, 1, N_BLK), jnp.int32),
    )(x, w, a, y)


def _sc_gather(table, idx):
    info = plsc.get_sparse_core_info()
    nc, ns = info.num_cores, info.num_subcores
    nw = nc * ns
    b_per_w = N // nw
    mesh = plsc.VectorSubcoreMesh(core_axis_name="c", subcore_axis_name="s")

    @functools.partial(
        pl.kernel,
        mesh=mesh,
        out_type=jax.ShapeDtypeStruct((N, U_COLS), jnp.float32),
        scratch_types=[
            pltpu.VMEM((b_per_w,), jnp.int32),
            pltpu.VMEM((b_per_w, U_COLS), jnp.float32),
            pltpu.SemaphoreType.DMA,
        ],
    )
    def gather_kernel(table_hbm, idx_hbm, out_hbm, idx_v, rows_v, sem):
        wid = lax.axis_index("s") * nc + lax.axis_index("c")
        base = wid * b_per_w
        pltpu.sync_copy(idx_hbm.at[pl.ds(base, b_per_w)], idx_v)
        pltpu.async_copy(table_hbm.at[idx_v], rows_v, sem).wait()
        pltpu.sync_copy(rows_v, out_hbm.at[pl.ds(base, b_per_w)])

    return gather_kernel(table, idx)


def kernel(y, x, B, A, u_grid):
    w = jnp.transpose(B, (2, 1, 0))
    w = jnp.pad(w, ((0, 0), (0, 0), (0, K_PAD - K)))
    w = w.reshape(D_X, D_Y * K_PAD)
    a = jnp.pad(A.T, ((0, 0), (0, K_PAD - K)), constant_values=A_PAD)
    a = a.reshape(1, D_Y * K_PAD)

    idx = _argmin_indices(x, w, a, y).reshape(N)

    table = jnp.pad(u_grid, ((0, 0), (0, U_COLS - D_Y)))
    rows = _sc_gather(table, idx)
    return rows[:, :D_Y]


# speedup vs baseline: 1.3461x; 1.0086x over previous
import functools

import jax
import jax.numpy as jnp
from jax import lax
from jax.experimental import pallas as pl
from jax.experimental.pallas import tpu as pltpu
from jax.experimental.pallas import tpu_sc as plsc

N = 4096
D_X = 128
D_Y = 2
K = 2500
K_PAD = 2560
N_BLK = 512
N_GRID = N // N_BLK
A_PAD = 2.0e18
U_COLS = 128


def _dist_argmin_body(x_ref, w_ref, a_ref, y_ref, idx_ref):
    s = jnp.dot(x_ref[...], w_ref[...], preferred_element_type=jnp.float32)
    s = s + a_ref[...]
    d0 = s[:, :K_PAD] - y_ref[:, 0:1]
    d1 = s[:, K_PAD:] - y_ref[:, 1:2]
    d2 = d0 * d0 + d1 * d1
    m = jnp.min(d2, axis=1, keepdims=True)
    k_iota = lax.broadcasted_iota(jnp.int32, d2.shape, 1)
    idx = jnp.min(jnp.where(d2 == m, k_iota, K_PAD), axis=1)
    idx_ref[0, 0, :] = idx


def _argmin_indices(x, w, a, y):
    return pl.pallas_call(
        _dist_argmin_body,
        grid=(N_GRID,),
        in_specs=[
            pl.BlockSpec((N_BLK, D_X), lambda i: (i, 0)),
            pl.BlockSpec((D_X, 2 * K_PAD), lambda i: (0, 0)),
            pl.BlockSpec((1, 2 * K_PAD), lambda i: (0, 0)),
            pl.BlockSpec((N_BLK, D_Y), lambda i: (i, 0)),
        ],
        out_specs=pl.BlockSpec((1, 1, N_BLK), lambda i: (i, 0, 0)),
        out_shape=jax.ShapeDtypeStruct((N_GRID, 1, N_BLK), jnp.int32),
    )(x, w, a, y)


def _sc_gather(table, idx):
    info = plsc.get_sparse_core_info()
    nc, ns = info.num_cores, info.num_subcores
    nw = nc * ns
    b_per_w = N // nw
    mesh = plsc.VectorSubcoreMesh(core_axis_name="c", subcore_axis_name="s")

    @functools.partial(
        pl.kernel,
        mesh=mesh,
        out_type=jax.ShapeDtypeStruct((N, U_COLS), jnp.float32),
        scratch_types=[
            pltpu.VMEM((b_per_w,), jnp.int32),
            pltpu.VMEM((b_per_w, U_COLS), jnp.float32),
            pltpu.SemaphoreType.DMA,
        ],
    )
    def gather_kernel(table_hbm, idx_hbm, out_hbm, idx_v, rows_v, sem):
        wid = lax.axis_index("s") * nc + lax.axis_index("c")
        base = wid * b_per_w
        pltpu.sync_copy(idx_hbm.at[pl.ds(base, b_per_w)], idx_v)
        pltpu.async_copy(table_hbm.at[idx_v], rows_v, sem).wait()
        pltpu.sync_copy(rows_v, out_hbm.at[pl.ds(base, b_per_w)])

    return gather_kernel(table, idx)


def kernel(y, x, B, A, u_grid):
    w = jnp.transpose(B, (2, 1, 0))
    w = jnp.pad(w, ((0, 0), (0, 0), (0, K_PAD - K)))
    w = w.reshape(D_X, D_Y * K_PAD)
    a = jnp.pad(A.T, ((0, 0), (0, K_PAD - K)), constant_values=A_PAD)
    a = a.reshape(1, D_Y * K_PAD)

    idx = _argmin_indices(x, w, a, y).reshape(N)

    table = jnp.pad(u_grid, ((0, 0), (0, U_COLS - D_Y)))
    return _sc_gather(table, idx)[:, :D_Y]
